# baseline (speedup 1.0000x reference)
import functools

import numpy as np
import jax
import jax.numpy as jnp
from jax import lax
from jax.experimental import pallas as pl
from jax.experimental.pallas import tpu as pltpu
from jax.experimental.pallas import tpu_sc as plsc

_C, _T, _H, _W = 3, 32, 384, 384
_ALPHA = 4
_NSLOW = _T // _ALPHA
_IDX = tuple(int(v) for v in np.linspace(0.0, _T - 1, _NSLOW).astype(np.float32))

_NWORKERS = 32
_ROWS = 128
_NCHUNKS = _H // _ROWS
_JOBS = tuple(
    (c, s, k) for c in range(_C) for s in range(_NSLOW) for k in range(_NCHUNKS)
)


def _sc_gather_body(frames_hbm, slow_hbm, buf):
    cid = lax.axis_index("c")
    sid = lax.axis_index("s")
    wid = sid * 2 + cid

    for j, (ch, slot, k) in enumerate(_JOBS):
        @pl.when(wid == j % _NWORKERS)
        def _(ch=ch, slot=slot, k=k):
            t = _IDX[slot]
            pltpu.sync_copy(frames_hbm.at[ch, t, pl.ds(k * _ROWS, _ROWS)], buf)
            pltpu.sync_copy(buf, slow_hbm.at[ch, slot, pl.ds(k * _ROWS, _ROWS)])


_sc_gather = functools.partial(
    pl.kernel,
    mesh=plsc.VectorSubcoreMesh(core_axis_name="c", subcore_axis_name="s"),
    out_type=jax.ShapeDtypeStruct((_C, _NSLOW, _H, _W), jnp.float32),
    scratch_types=[pltpu.VMEM((_ROWS, _W), jnp.float32)],
)(_sc_gather_body)

_OCT = 8


def _tc_copy_body(in_ref, fast_ref):
    fast_ref[...] = in_ref[...]


def _tc_copy(frames):
    return pl.pallas_call(
        _tc_copy_body,
        grid=(_T // _OCT,),
        in_specs=[pl.BlockSpec((_C, _OCT, _H, _W), lambda o: (0, o, 0, 0))],
        out_specs=pl.BlockSpec((_C, _OCT, _H, _W), lambda o: (0, o, 0, 0)),
        out_shape=jax.ShapeDtypeStruct((_C, _T, _H, _W), frames.dtype),
        compiler_params=pltpu.CompilerParams(
            vmem_limit_bytes=100 * 1024 * 1024,
        ),
    )(frames)


def kernel(frames):
    fast = _tc_copy(frames)
    slow = _sc_gather(frames)
    return (slow, fast)


# speedup vs baseline: 1.5435x; 1.5435x over previous
import numpy as np
import jax
import jax.numpy as jnp
from jax.experimental import pallas as pl
from jax.experimental.pallas import tpu as pltpu

_C, _T, _H, _W = 3, 32, 384, 384
_ALPHA = 4
_NSLOW = _T // _ALPHA
_IDX = tuple(int(v) for v in np.linspace(0.0, _T - 1, _NSLOW).astype(np.float32))
_OCT = 8
assert all(_IDX[2 * o] - _OCT * o == max(0, o - 1) for o in range(_T // _OCT))
assert all(_IDX[2 * o + 1] - _OCT * o == o + 4 for o in range(_T // _OCT))


def _body(in_ref, slow_ref, fast_ref):
    o = pl.program_id(0)
    fast_ref[...] = in_ref[...]
    off0 = jnp.maximum(0, o - 1)
    off1 = o + 4
    slow_ref[:, pl.ds(0, 1)] = in_ref[:, pl.ds(off0, 1)]
    slow_ref[:, pl.ds(1, 1)] = in_ref[:, pl.ds(off1, 1)]


def kernel(frames):
    slow, fast = pl.pallas_call(
        _body,
        grid=(_T // _OCT,),
        in_specs=[pl.BlockSpec((_C, _OCT, _H, _W), lambda o: (0, o, 0, 0))],
        out_specs=[
            pl.BlockSpec((_C, 2, _H, _W), lambda o: (0, o, 0, 0)),
            pl.BlockSpec((_C, _OCT, _H, _W), lambda o: (0, o, 0, 0)),
        ],
        out_shape=[
            jax.ShapeDtypeStruct((_C, _NSLOW, _H, _W), frames.dtype),
            jax.ShapeDtypeStruct((_C, _T, _H, _W), frames.dtype),
        ],
        compiler_params=pltpu.CompilerParams(
            vmem_limit_bytes=100 * 1024 * 1024,
        ),
    )(frames)
    return (slow, fast)
